# baseline (speedup 1.0000x reference)
import functools

import jax
import jax.numpy as jnp
import numpy as np
from jax import lax
from jax.experimental import pallas as pl
from jax.experimental.pallas import tpu as pltpu
from jax.experimental.pallas import tpu_sc as plsc

_LANES = 16


def _interleave_perm(D):
    q = np.empty(32, np.int32)
    q[0::2] = np.arange(16)
    q[1::2] = np.arange(16, 32)
    return np.concatenate([c * 32 + q for c in range(D // 32)])


@functools.partial(jax.jit, static_argnums=(4, 5, 6))
def _sc_fused(idx, posd, w2, t2, N, D, A):
    V = t2.shape[0] * 2 // D
    NW = 32
    RPW = N // NW
    R = 320
    NBLK = RPW // R
    C = D // 32
    assert RPW % R == 0 and N % NW == 0 and NBLK % 2 == 0

    mesh = plsc.VectorSubcoreMesh(core_axis_name="c", subcore_axis_name="s")

    @functools.partial(
        pl.kernel,
        mesh=mesh,
        compiler_params=pltpu.CompilerParams(needs_layout_passes=False),
        out_type=jax.ShapeDtypeStruct((N * D,), jnp.float32),
        scratch_types=[
            pltpu.VMEM((2 * R,), jnp.int32),
            pltpu.VMEM((2 * R * A,), jnp.uint32),
            pltpu.VMEM((2 * R * D,), jnp.float32),
            pltpu.VMEM((V * D // 2,), jnp.uint32),
            pltpu.VMEM((A * D // 2,), jnp.uint32),
            pltpu.SemaphoreType.DMA,
            pltpu.SemaphoreType.DMA,
            pltpu.SemaphoreType.DMA,
            pltpu.SemaphoreType.DMA,
        ],
    )
    def k(idx_hbm, posd_hbm, w2_hbm, t2_hbm, out_hbm, idx_v, posd_v, out_v,
          t2_v, w2_v, sin0, sin1, sout0, sout1):
        sins = (sin0, sin1)
        souts = (sout0, sout1)
        wid = lax.axis_index("s") * 2 + lax.axis_index("c")
        base = wid * RPW
        pltpu.sync_copy(t2_hbm, t2_v)
        pltpu.sync_copy(w2_hbm, w2_v)
        w2regs = [
            [
                plsc.bitcast(
                    w2_v[pl.ds(a * (D // 2) + c * _LANES, _LANES)], jnp.bfloat16
                )
                for c in range(C)
            ]
            for a in range(A)
        ]

        def start_in(j, b):
            row0 = base + j * R
            pltpu.async_copy(idx_hbm.at[pl.ds(row0, R)], idx_v.at[pl.ds(b * R, R)], sins[b])
            for a in range(A):
                pltpu.async_copy(
                    posd_hbm.at[pl.ds(a * N + row0, R)],
                    posd_v.at[pl.ds((b * A + a) * R, R)],
                    sins[b],
                )

        def wait_in(b):
            pltpu.make_async_copy(
                idx_hbm.at[pl.ds(0, R)], idx_v.at[pl.ds(b * R, R)], sins[b]
            ).wait()
            pltpu.make_async_copy(
                posd_hbm.at[pl.ds(0, R * A)], posd_v.at[pl.ds(b * R * A, R * A)], sins[b]
            ).wait()

        def wait_out(b):
            pltpu.make_async_copy(
                out_v.at[pl.ds(b * R * D, R * D)], out_hbm.at[pl.ds(0, R * D)], souts[b]
            ).wait()

        def compute(j, b):
            ibase = b * R
            pbase = b * R * A
            obase = b * R * D

            @plsc.parallel_loop(0, R, unroll=8)
            def row1(r):
                idxv = idx_v[pl.ds(ibase + r, _LANES)] * (D // 2)
                tb = idxv[0]
                bcs = [
                    plsc.bitcast(
                        jnp.broadcast_to(
                            posd_v[pl.ds(pbase + a * R + r, _LANES)][0], (_LANES,)
                        ),
                        jnp.bfloat16,
                    )
                    for a in range(A)
                ]
                accs = [
                    plsc.bitcast(
                        t2_v[pl.ds(tb + c * _LANES, _LANES)], jnp.bfloat16
                    )
                    for c in range(C)
                ]
                for a in range(A):
                    accs = [accs[c] + bcs[a] * w2regs[a][c] for c in range(C)]
                ob = obase + r * D
                for c in range(C):
                    lo, hi = plsc.unpack(
                        accs[c], format=plsc.PackFormat.INTERLEAVED
                    )
                    out_v[pl.ds(ob + c * 32, _LANES)] = lo
                    out_v[pl.ds(ob + c * 32 + _LANES, _LANES)] = hi

        start_in(0, 0)
        start_in(1, 1)

        def dbl(j2, carry):
            for b in range(2):
                j = 2 * j2 + b
                wait_in(b)

                @pl.when(j2 > 0)
                def _():
                    wait_out(b)

                compute(j, b)
                row0 = base + j * R
                pltpu.async_copy(
                    out_v.at[pl.ds(b * R * D, R * D)], out_hbm.at[pl.ds(row0 * D, R * D)], souts[b]
                )

                @pl.when(j2 < NBLK // 2 - 1)
                def _():
                    start_in(j + 2, b)

            return carry

        lax.fori_loop(0, NBLK // 2, dbl, 0)
        wait_out(0)
        wait_out(1)

    return k(idx, posd, w2, t2)


def kernel(command_indices_tensor, positions_tensor, command_table, lin_w, lin_b):
    B, L = command_indices_tensor.shape
    V, D = command_table.shape
    A = positions_tensor.shape[-1]
    N = B * L
    scale = jnp.float32(D) ** 0.5

    def _pack_words(m):
        u = lax.bitcast_convert_type(
            m[:, _interleave_perm(D)].astype(jnp.bfloat16), jnp.uint16
        ).astype(jnp.uint32)
        return (u[:, 0::2] | (u[:, 1::2] << 16)).reshape(-1)

    w2 = _pack_words((lin_w * scale).T)
    t2 = _pack_words((command_table + lin_b[None, :]) * scale)
    idx = command_indices_tensor.reshape(N).astype(jnp.int32)
    pb = lax.bitcast_convert_type(
        jnp.transpose(positions_tensor.astype(jnp.bfloat16), (2, 0, 1)).reshape(
            A * N
        ),
        jnp.uint16,
    ).astype(jnp.uint32)
    posd = pb | (pb << 16)
    out = _sc_fused(idx, posd, w2, t2, N, D, A)
    return out.reshape(B, L, D)


# speedup vs baseline: 1.2491x; 1.2491x over previous
import functools

import jax
import jax.numpy as jnp
import numpy as np
from jax import lax
from jax.experimental import pallas as pl
from jax.experimental.pallas import tpu as pltpu
from jax.experimental.pallas import tpu_sc as plsc

_LANES = 16


def _interleave_perm(D):
    q = np.empty(32, np.int32)
    q[0::2] = np.arange(16)
    q[1::2] = np.arange(16, 32)
    return np.concatenate([c * 32 + q for c in range(D // 32)])


@functools.partial(jax.jit, static_argnums=(4, 5, 6))
def _sc_fused(idx, posd, w2, t2, N, D, A):
    V = t2.shape[0] * 2 // D
    NW = 32
    RPW = N // NW
    R = 320
    NBLK = RPW // R
    C = D // 32
    assert RPW % R == 0 and N % NW == 0 and NBLK % 2 == 0

    mesh = plsc.VectorSubcoreMesh(core_axis_name="c", subcore_axis_name="s")

    @functools.partial(
        pl.kernel,
        mesh=mesh,
        compiler_params=pltpu.CompilerParams(needs_layout_passes=False),
        out_type=jax.ShapeDtypeStruct((N * D,), jnp.float32),
        scratch_types=[
            pltpu.VMEM((2 * R,), jnp.int32),
            pltpu.VMEM((2 * R * A,), jnp.uint32),
            pltpu.VMEM((2 * R * D,), jnp.float32),
            pltpu.VMEM((V * D // 2,), jnp.uint32),
            pltpu.VMEM((A * D // 2,), jnp.uint32),
            pltpu.SemaphoreType.DMA,
            pltpu.SemaphoreType.DMA,
            pltpu.SemaphoreType.DMA,
            pltpu.SemaphoreType.DMA,
        ],
    )
    def k(idx_hbm, posd_hbm, w2_hbm, t2_hbm, out_hbm, idx_v, posd_v, out_v,
          t2_v, w2_v, sin0, sin1, sout0, sout1):
        sins = (sin0, sin1)
        souts = (sout0, sout1)
        wid = lax.axis_index("s") * 2 + lax.axis_index("c")
        base = wid * RPW
        pltpu.sync_copy(t2_hbm, t2_v)
        pltpu.sync_copy(w2_hbm, w2_v)
        w2regs = [
            [
                plsc.bitcast(
                    w2_v[pl.ds(a * (D // 2) + c * _LANES, _LANES)], jnp.bfloat16
                )
                for c in range(C)
            ]
            for a in range(A)
        ]

        def start_in(j, b):
            row0 = base + j * R
            pltpu.async_copy(idx_hbm.at[pl.ds(row0, R)], idx_v.at[pl.ds(b * R, R)], sins[b])
            for a in range(A):
                pltpu.async_copy(
                    posd_hbm.at[pl.ds(a * N + row0, R)],
                    posd_v.at[pl.ds((b * A + a) * R, R)],
                    sins[b],
                )

        def wait_in(b):
            pltpu.make_async_copy(
                idx_hbm.at[pl.ds(0, R)], idx_v.at[pl.ds(b * R, R)], sins[b]
            ).wait()
            pltpu.make_async_copy(
                posd_hbm.at[pl.ds(0, R * A)], posd_v.at[pl.ds(b * R * A, R * A)], sins[b]
            ).wait()

        def wait_out(b):
            pltpu.make_async_copy(
                out_v.at[pl.ds(b * R * D, R * D)], out_hbm.at[pl.ds(0, R * D)], souts[b]
            ).wait()

        def compute(j, b):
            ibase = b * R
            pbase = b * R * A
            obase = b * R * D

            @plsc.parallel_loop(0, R, unroll=6)
            def row1(r):
                idxv = idx_v[pl.ds(ibase + r, _LANES)] * (D // 2)
                tb = idxv[0]
                bcs = [
                    plsc.bitcast(
                        jnp.broadcast_to(
                            posd_v[pl.ds(pbase + a * R + r, _LANES)][0], (_LANES,)
                        ),
                        jnp.bfloat16,
                    )
                    for a in range(A)
                ]
                accs = [
                    plsc.bitcast(
                        t2_v[pl.ds(tb + c * _LANES, _LANES)], jnp.bfloat16
                    )
                    for c in range(C)
                ]
                for a in range(A):
                    accs = [accs[c] + bcs[a] * w2regs[a][c] for c in range(C)]
                ob = obase + r * D
                for c in range(C):
                    lo, hi = plsc.unpack(
                        accs[c], format=plsc.PackFormat.INTERLEAVED
                    )
                    out_v[pl.ds(ob + c * 32, _LANES)] = lo
                    out_v[pl.ds(ob + c * 32 + _LANES, _LANES)] = hi

        start_in(0, 0)
        start_in(1, 1)

        def dbl(j2, carry):
            for b in range(2):
                j = 2 * j2 + b
                wait_in(b)

                @pl.when(j2 > 0)
                def _():
                    wait_out(b)

                compute(j, b)
                row0 = base + j * R
                pltpu.async_copy(
                    out_v.at[pl.ds(b * R * D, R * D)], out_hbm.at[pl.ds(row0 * D, R * D)], souts[b]
                )

                @pl.when(j2 < NBLK // 2 - 1)
                def _():
                    start_in(j + 2, b)

            return carry

        lax.fori_loop(0, NBLK // 2, dbl, 0)
        wait_out(0)
        wait_out(1)

    return k(idx, posd, w2, t2)


def kernel(command_indices_tensor, positions_tensor, command_table, lin_w, lin_b):
    B, L = command_indices_tensor.shape
    V, D = command_table.shape
    A = positions_tensor.shape[-1]
    N = B * L
    scale = jnp.float32(D) ** 0.5

    def _pack_words(m):
        u = lax.bitcast_convert_type(
            m[:, _interleave_perm(D)].astype(jnp.bfloat16), jnp.uint16
        ).astype(jnp.uint32)
        return (u[:, 0::2] | (u[:, 1::2] << 16)).reshape(-1)

    w2 = _pack_words((lin_w * scale).T)
    t2 = _pack_words((command_table + lin_b[None, :]) * scale)
    idx = command_indices_tensor.reshape(N).astype(jnp.int32)
    pb = lax.bitcast_convert_type(
        jnp.transpose(positions_tensor.astype(jnp.bfloat16), (2, 0, 1)).reshape(
            A * N
        ),
        jnp.uint16,
    ).astype(jnp.uint32)
    posd = pb | (pb << 16)
    out = _sc_fused(idx, posd, w2, t2, N, D, A)
    return out.reshape(B, L, D)


# speedup vs baseline: 1.3746x; 1.1005x over previous
import functools

import jax
import jax.numpy as jnp
import numpy as np
from jax import lax
from jax.experimental import pallas as pl
from jax.experimental.pallas import tpu as pltpu
from jax.experimental.pallas import tpu_sc as plsc

_LANES = 16


def _interleave_perm(D):
    q = np.empty(32, np.int32)
    q[0::2] = np.arange(16)
    q[1::2] = np.arange(16, 32)
    return np.concatenate([c * 32 + q for c in range(D // 32)])


@functools.partial(jax.jit, static_argnums=(4, 5, 6))
def _sc_fused(idx, posd, w2, t2, N, D, A):
    V = t2.shape[0] * 2 // D
    NW = 32
    RPW = N // NW
    R = 320
    NBLK = RPW // R
    C = D // 32
    assert RPW % R == 0 and N % NW == 0 and NBLK % 2 == 0

    mesh = plsc.VectorSubcoreMesh(core_axis_name="c", subcore_axis_name="s")

    @functools.partial(
        pl.kernel,
        mesh=mesh,
        compiler_params=pltpu.CompilerParams(needs_layout_passes=False),
        out_type=jax.ShapeDtypeStruct((N * D,), jnp.float32),
        scratch_types=[
            pltpu.VMEM((2 * R,), jnp.int32),
            pltpu.VMEM((2 * R * A,), jnp.uint32),
            pltpu.VMEM((2 * R * D,), jnp.float32),
            pltpu.VMEM((V * D // 2,), jnp.uint32),
            pltpu.VMEM((A * D // 2,), jnp.uint32),
            pltpu.SemaphoreType.DMA,
            pltpu.SemaphoreType.DMA,
            pltpu.SemaphoreType.DMA,
            pltpu.SemaphoreType.DMA,
        ],
    )
    def k(idx_hbm, posd_hbm, w2_hbm, t2_hbm, out_hbm, idx_v, posd_v, out_v,
          t2_v, w2_v, sin0, sin1, sout0, sout1):
        sins = (sin0, sin1)
        souts = (sout0, sout1)
        wid = lax.axis_index("s") * 2 + lax.axis_index("c")
        base = wid * RPW
        pltpu.sync_copy(t2_hbm, t2_v)
        pltpu.sync_copy(w2_hbm, w2_v)
        w2regs = [
            [
                plsc.bitcast(
                    w2_v[pl.ds(a * (D // 2) + c * _LANES, _LANES)], jnp.bfloat16
                )
                for c in range(C)
            ]
            for a in range(A)
        ]

        def start_in(j, b):
            row0 = base + j * R
            pltpu.async_copy(idx_hbm.at[pl.ds(row0, R)], idx_v.at[pl.ds(b * R, R)], sins[b])
            for a in range(A):
                pltpu.async_copy(
                    posd_hbm.at[pl.ds(a * N + row0, R)],
                    posd_v.at[pl.ds((b * A + a) * R, R)],
                    sins[b],
                )

        def wait_in(b):
            pltpu.make_async_copy(
                idx_hbm.at[pl.ds(0, R)], idx_v.at[pl.ds(b * R, R)], sins[b]
            ).wait()
            pltpu.make_async_copy(
                posd_hbm.at[pl.ds(0, R * A)], posd_v.at[pl.ds(b * R * A, R * A)], sins[b]
            ).wait()

        def wait_out(b):
            pltpu.make_async_copy(
                out_v.at[pl.ds(b * R * D, R * D)], out_hbm.at[pl.ds(0, R * D)], souts[b]
            ).wait()

        def compute(j, b):
            ibase = b * R
            pbase = b * R * A
            obase = b * R * D

            @plsc.parallel_loop(0, R, unroll=4)
            def row1(r):
                idxv = idx_v[pl.ds(ibase + r, _LANES)] * (D // 2)
                tb = idxv[0]
                bcs = [
                    plsc.bitcast(
                        jnp.broadcast_to(
                            posd_v[pl.ds(pbase + a * R + r, _LANES)][0], (_LANES,)
                        ),
                        jnp.bfloat16,
                    )
                    for a in range(A)
                ]
                accs = [
                    plsc.bitcast(
                        t2_v[pl.ds(tb + c * _LANES, _LANES)], jnp.bfloat16
                    )
                    for c in range(C)
                ]
                for a in range(A):
                    accs = [accs[c] + bcs[a] * w2regs[a][c] for c in range(C)]
                ob = obase + r * D
                for c in range(C):
                    lo, hi = plsc.unpack(
                        accs[c], format=plsc.PackFormat.INTERLEAVED
                    )
                    out_v[pl.ds(ob + c * 32, _LANES)] = lo
                    out_v[pl.ds(ob + c * 32 + _LANES, _LANES)] = hi

        start_in(0, 0)
        start_in(1, 1)

        def dbl(j2, carry):
            for b in range(2):
                j = 2 * j2 + b
                wait_in(b)

                @pl.when(j2 > 0)
                def _():
                    wait_out(b)

                compute(j, b)
                row0 = base + j * R
                pltpu.async_copy(
                    out_v.at[pl.ds(b * R * D, R * D)], out_hbm.at[pl.ds(row0 * D, R * D)], souts[b]
                )

                @pl.when(j2 < NBLK // 2 - 1)
                def _():
                    start_in(j + 2, b)

            return carry

        lax.fori_loop(0, NBLK // 2, dbl, 0)
        wait_out(0)
        wait_out(1)

    return k(idx, posd, w2, t2)


def kernel(command_indices_tensor, positions_tensor, command_table, lin_w, lin_b):
    B, L = command_indices_tensor.shape
    V, D = command_table.shape
    A = positions_tensor.shape[-1]
    N = B * L
    scale = jnp.float32(D) ** 0.5

    def _pack_words(m):
        u = lax.bitcast_convert_type(
            m[:, _interleave_perm(D)].astype(jnp.bfloat16), jnp.uint16
        ).astype(jnp.uint32)
        return (u[:, 0::2] | (u[:, 1::2] << 16)).reshape(-1)

    w2 = _pack_words((lin_w * scale).T)
    t2 = _pack_words((command_table + lin_b[None, :]) * scale)
    idx = command_indices_tensor.reshape(N).astype(jnp.int32)
    pb = lax.bitcast_convert_type(
        jnp.transpose(positions_tensor.astype(jnp.bfloat16), (2, 0, 1)).reshape(
            A * N
        ),
        jnp.uint16,
    ).astype(jnp.uint32)
    posd = pb | (pb << 16)
    out = _sc_fused(idx, posd, w2, t2, N, D, A)
    return out.reshape(B, L, D)
